# baseline (speedup 1.0000x reference)
import math

import jax
import jax.numpy as jnp
from jax.experimental import pallas as pl
from jax.experimental.pallas import tpu as pltpu

_LN_EPS = 1e-12
_SQRT_HALF = 1.0 / math.sqrt(2.0)


def _cdiv(a, b):
    return (a + b - 1) // b


def _transform_kernel(x_ref, w1_ref, b1_ref, gamma_ref, beta_ref, h_ref):
    x = x_ref[...].astype(jnp.bfloat16)
    w = w1_ref[...].astype(jnp.bfloat16)
    y = jax.lax.dot_general(x, w, (((1,), (1,)), ((), ())),
                            preferred_element_type=jnp.float32)
    y = y + b1_ref[...]
    y = y * 0.5 * (1.0 + jax.lax.erf(y * _SQRT_HALF))
    mean = jnp.mean(y, axis=-1, keepdims=True)
    centered = y - mean
    var = jnp.mean(centered * centered, axis=-1, keepdims=True)
    y = centered * jax.lax.rsqrt(var + _LN_EPS)
    y = y * gamma_ref[...] + beta_ref[...]
    h_ref[...] = y.astype(h_ref.dtype)


def _decoder_kernel(h_ref, wv_ref, bv_ref, out_ref):
    w = wv_ref[...].astype(jnp.bfloat16)
    logits = jax.lax.dot_general(h_ref[...], w, (((1,), (1,)), ((), ())),
                                 preferred_element_type=jnp.float32)
    out_ref[...] = logits + bv_ref[...]


def kernel(x, w1, b1, gamma, beta, dec_w, dec_b):
    B, S, H = x.shape
    V = dec_w.shape[0]
    M = B * S

    x2 = x.reshape(M, H)
    b1_2 = b1.reshape(1, H).astype(jnp.float32)
    gamma_2 = gamma.reshape(1, H).astype(jnp.float32)
    beta_2 = beta.reshape(1, H).astype(jnp.float32)
    dec_b_2 = dec_b.reshape(1, V).astype(jnp.float32)

    tm = min(512, M)
    h = pl.pallas_call(
        _transform_kernel,
        out_shape=jax.ShapeDtypeStruct((M, H), jnp.bfloat16),
        grid=(_cdiv(M, tm),),
        in_specs=[
            pl.BlockSpec((tm, H), lambda i: (i, 0)),
            pl.BlockSpec((H, H), lambda i: (0, 0)),
            pl.BlockSpec((1, H), lambda i: (0, 0)),
            pl.BlockSpec((1, H), lambda i: (0, 0)),
            pl.BlockSpec((1, H), lambda i: (0, 0)),
        ],
        out_specs=pl.BlockSpec((tm, H), lambda i: (i, 0)),
        compiler_params=pltpu.CompilerParams(
            dimension_semantics=("parallel",),
            vmem_limit_bytes=64 * 1024 * 1024,
        ),
        cost_estimate=pl.CostEstimate(
            flops=2 * M * H * H,
            transcendentals=M * H,
            bytes_accessed=4 * (M * H + H * H + 3 * H) + 2 * M * H,
        ),
    )(x2, w1, b1_2, gamma_2, beta_2)

    tv = 1280
    njt = _cdiv(V, tv)
    n_seg = 2 if njt >= 4 else 1
    seg_tiles = _cdiv(njt, n_seg)
    parts = []
    col = 0
    for s in range(n_seg):
        t0 = s * seg_tiles
        nt = min(seg_tiles, njt - t0)
        seg_cols = min(nt * tv, V - col)

        def _mk(off):
            return (lambda j: (j + off, 0)), (lambda j: (0, j + off))

        w_map, c_map = _mk(t0)
        part = pl.pallas_call(
            _decoder_kernel,
            out_shape=jax.ShapeDtypeStruct((M, nt * tv), jnp.float32),
            grid=(nt,),
            in_specs=[
                pl.BlockSpec((M, H), lambda j: (0, 0)),
                pl.BlockSpec((tv, H), w_map),
                pl.BlockSpec((1, tv), c_map),
            ],
            out_specs=pl.BlockSpec((M, tv), lambda j: (0, j)),
            compiler_params=pltpu.CompilerParams(
                dimension_semantics=("parallel",),
                vmem_limit_bytes=64 * 1024 * 1024,
            ),
            cost_estimate=pl.CostEstimate(
                flops=2 * M * H * nt * tv,
                transcendentals=0,
                bytes_accessed=2 * M * H + 4 * (H * nt * tv + nt * tv
                                                + M * nt * tv),
            ),
        )(h, dec_w, dec_b_2)
        parts.append(part[:, :seg_cols])
        col += seg_cols

    out = parts[0] if n_seg == 1 else jnp.concatenate(parts, axis=1)
    return out.reshape(B, S, V)


# speedup vs baseline: 1.8134x; 1.8134x over previous
import math

import jax
import jax.numpy as jnp
from jax.experimental import pallas as pl
from jax.experimental.pallas import tpu as pltpu

_LN_EPS = 1e-12
_SQRT_HALF = 1.0 / math.sqrt(2.0)


def _cdiv(a, b):
    return (a + b - 1) // b


def _transform_kernel(x_ref, w1_ref, b1_ref, gamma_ref, beta_ref, h_ref):
    x = x_ref[...].astype(jnp.bfloat16)
    w = w1_ref[...].astype(jnp.bfloat16)
    y = jax.lax.dot_general(x, w, (((1,), (1,)), ((), ())),
                            preferred_element_type=jnp.float32)
    y = y + b1_ref[...]
    y = y * 0.5 * (1.0 + jax.lax.erf(y * _SQRT_HALF))
    mean = jnp.mean(y, axis=-1, keepdims=True)
    centered = y - mean
    var = jnp.mean(centered * centered, axis=-1, keepdims=True)
    y = centered * jax.lax.rsqrt(var + _LN_EPS)
    y = y * gamma_ref[...] + beta_ref[...]
    h_ref[...] = y.astype(h_ref.dtype)


def _decoder_kernel(h_ref, wv_ref, bv_ref, out_ref):
    w = wv_ref[...].astype(jnp.bfloat16)
    logits = jax.lax.dot_general(h_ref[...], w, (((1,), (1,)), ((), ())),
                                 preferred_element_type=jnp.float32)
    out_ref[...] = logits + bv_ref[...]


def kernel(x, w1, b1, gamma, beta, dec_w, dec_b):
    B, S, H = x.shape
    V = dec_w.shape[0]
    M = B * S

    x2 = x.reshape(M, H)
    b1_2 = b1.reshape(1, H).astype(jnp.float32)
    gamma_2 = gamma.reshape(1, H).astype(jnp.float32)
    beta_2 = beta.reshape(1, H).astype(jnp.float32)
    dec_b_2 = dec_b.reshape(1, V).astype(jnp.float32)

    tm = min(1024, M)
    h = pl.pallas_call(
        _transform_kernel,
        out_shape=jax.ShapeDtypeStruct((M, H), jnp.bfloat16),
        grid=(_cdiv(M, tm),),
        in_specs=[
            pl.BlockSpec((tm, H), lambda i: (i, 0)),
            pl.BlockSpec((H, H), lambda i: (0, 0)),
            pl.BlockSpec((1, H), lambda i: (0, 0)),
            pl.BlockSpec((1, H), lambda i: (0, 0)),
            pl.BlockSpec((1, H), lambda i: (0, 0)),
        ],
        out_specs=pl.BlockSpec((tm, H), lambda i: (i, 0)),
        compiler_params=pltpu.CompilerParams(
            dimension_semantics=("parallel",),
            vmem_limit_bytes=64 * 1024 * 1024,
        ),
        cost_estimate=pl.CostEstimate(
            flops=2 * M * H * H,
            transcendentals=M * H,
            bytes_accessed=4 * (M * H + H * H + 3 * H) + 2 * M * H,
        ),
    )(x2, w1, b1_2, gamma_2, beta_2)

    tv = 1280
    out = pl.pallas_call(
        _decoder_kernel,
        out_shape=jax.ShapeDtypeStruct((M, V), jnp.float32),
        grid=(_cdiv(V, tv),),
        in_specs=[
            pl.BlockSpec((M, H), lambda j: (0, 0)),
            pl.BlockSpec((tv, H), lambda j: (j, 0)),
            pl.BlockSpec((1, tv), lambda j: (0, j)),
        ],
        out_specs=pl.BlockSpec((M, tv), lambda j: (0, j)),
        compiler_params=pltpu.CompilerParams(
            dimension_semantics=("parallel",),
            vmem_limit_bytes=64 * 1024 * 1024,
        ),
        cost_estimate=pl.CostEstimate(
            flops=2 * M * H * V,
            transcendentals=0,
            bytes_accessed=2 * M * H + 4 * (H * V + V + M * V),
        ),
    )(h, dec_w, dec_b_2)

    return out.reshape(B, S, V)
